# baseline (speedup 1.0000x reference)
import functools

import jax
import jax.numpy as jnp
from jax import lax
from jax.experimental import pallas as pl
from jax.experimental.pallas import tpu as pltpu
from jax.experimental.pallas import tpu_sc as plsc

N_NODES = 10000
N_EDGES = 320000
D_IN = 128
D_H = 64

NC = 2
NS = 16
NW = NC * NS

CH = 256
NCHUNK = 40
EDGES_PER_W = CH * NCHUNK
E_PAD = EDGES_PER_W * NW
N_PAD = 10240
ROWS_PER_TILE = N_PAD // NS
N_ACC = 10112
ROWS_ACC = N_ACC // NS
DUMMY_DST = N_NODES

ZR = 79


def _sc_aggregate(with_deg):
    G = 1
    out_type = [jax.ShapeDtypeStruct((NC, N_NODES, D_H), jnp.float32)]
    scratch = (
        [pltpu.VMEM((2, 2 * G, CH), jnp.int32),
         pltpu.VMEM((2, 2 * G, CH), jnp.int32)]
        + [pltpu.VMEM((CH, D_H), jnp.float32)] * (2 * G)
        + [pltpu.VMEM((ZR, D_H), jnp.float32),
           pltpu.VMEM_SHARED((N_ACC, D_H), jnp.float32),
           pltpu.VMEM_SHARED((N_NODES, D_H), jnp.float32)]
        + [pltpu.SemaphoreType.DMA] * 3
    )
    if with_deg:
        out_type.append(jax.ShapeDtypeStruct((NC, 1, N_PAD), jnp.float32))
        scratch += [
            pltpu.VMEM((CH,), jnp.float32),
            pltpu.VMEM((ROWS_PER_TILE,), jnp.float32),
            pltpu.VMEM_SHARED((N_PAD,), jnp.float32),
            pltpu.SemaphoreType.DMA,
        ]

    mesh = plsc.VectorSubcoreMesh(core_axis_name="c", subcore_axis_name="s")

    @functools.partial(pl.kernel, out_type=out_type, mesh=mesh,
                       scratch_types=scratch,
                       compiler_params=pltpu.CompilerParams(
                           use_tc_tiling_on_sc=False))
    def body(src3, dst3, y_hbm, *rest):
        nb = 2 * G
        if with_deg:
            (agg_out, deg_out, src_i, dst_i, *rows, zrow_v, acc_sh, y_sh) = \
                rest[:7 + nb]
            gsem, ssem, isem, ones_v, zd_v, deg_sh, dsem = rest[7 + nb:]
        else:
            (agg_out, src_i, dst_i, *rows, zrow_v, acc_sh, y_sh) = \
                rest[:6 + nb]
            gsem, ssem, isem = rest[6 + nb:]

        cid = lax.axis_index("c")
        sid = lax.axis_index("s")
        wid = sid * NC + cid
        base_row = sid * ROWS_PER_TILE
        base_acc = sid * ROWS_ACC

        z16 = jnp.zeros((16,), jnp.float32)

        def zrow_body(r, _):
            for c4 in range(D_H // 16):
                zrow_v[r, pl.ds(c4 * 16, 16)] = z16
            return 0
        lax.fori_loop(0, ZR, zrow_body, 0)

        def zacc_body(k, _):
            pltpu.sync_copy(zrow_v, acc_sh.at[pl.ds(base_acc + k * ZR, ZR)])
            return 0
        lax.fori_loop(0, ROWS_ACC // ZR, zacc_body, 0)

        if with_deg:
            for i in range(CH // 16):
                ones_v[pl.ds(i * 16, 16)] = z16 + 1.0
            def zd_body(i, _):
                zd_v[pl.ds(i * 16, 16)] = z16
                return 0
            lax.fori_loop(0, ROWS_PER_TILE // 16, zd_body, 0)
            pltpu.sync_copy(zd_v, deg_sh.at[pl.ds(base_row, ROWS_PER_TILE)])

        pltpu.sync_copy(src3.at[wid, pl.ds(0, nb)], src_i.at[0])
        pltpu.sync_copy(dst3.at[wid, pl.ds(0, nb)], dst_i.at[0])

        nvalid_last = N_NODES - (NS - 1) * ROWS_PER_TILE

        @pl.when(sid < NS - 1)
        def _():
            pltpu.sync_copy(y_hbm.at[pl.ds(base_row, ROWS_PER_TILE)],
                            y_sh.at[pl.ds(base_row, ROWS_PER_TILE)])

        @pl.when(sid == NS - 1)
        def _():
            pltpu.sync_copy(y_hbm.at[pl.ds(base_row, nvalid_last)],
                            y_sh.at[pl.ds(base_row, nvalid_last)])

        plsc.subcore_barrier()

        def chunk_body(m, _):
            h = lax.rem(m, 2)
            j0 = nb * m
            ga = [pltpu.async_copy(y_sh.at[src_i.at[h, i]], rows[i], gsem)
                  for i in range(G)]
            nxt = jnp.minimum(j0 + nb, NCHUNK - nb)
            ia = pltpu.async_copy(src3.at[wid, pl.ds(nxt, nb)],
                                  src_i.at[1 - h], isem)
            ib = pltpu.async_copy(dst3.at[wid, pl.ds(nxt, nb)],
                                  dst_i.at[1 - h], isem)
            for cp in ga:
                cp.wait()
            gb = [pltpu.async_copy(y_sh.at[src_i.at[h, G + i]],
                                   rows[G + i], gsem) for i in range(G)]
            sa = [pltpu.async_copy(rows[i], acc_sh.at[dst_i.at[h, i]],
                                   ssem, add=True) for i in range(G)]
            if with_deg:
                da = [pltpu.async_copy(ones_v, deg_sh.at[dst_i.at[h, i]],
                                       dsem, add=True) for i in range(nb)]
            for cp in gb:
                cp.wait()
            sb = [pltpu.async_copy(rows[G + i],
                                   acc_sh.at[dst_i.at[h, G + i]],
                                   ssem, add=True) for i in range(G)]
            for cp in sa + sb:
                cp.wait()
            if with_deg:
                for cp in da:
                    cp.wait()
            ia.wait()
            ib.wait()
            return 0
        lax.fori_loop(0, NCHUNK // nb, chunk_body, 0)

        plsc.subcore_barrier()

        nacc_last = N_NODES - (NS - 1) * ROWS_ACC

        @pl.when(sid < NS - 1)
        def _():
            pltpu.sync_copy(acc_sh.at[pl.ds(base_acc, ROWS_ACC)],
                            agg_out.at[cid, pl.ds(base_acc, ROWS_ACC)])

        @pl.when(sid == NS - 1)
        def _():
            pltpu.sync_copy(acc_sh.at[pl.ds(base_acc, nacc_last)],
                            agg_out.at[cid, pl.ds(base_acc, nacc_last)])

        if with_deg:
            pltpu.sync_copy(deg_sh.at[pl.ds(base_row, ROWS_PER_TILE)],
                            deg_out.at[cid, 0, pl.ds(base_row, ROWS_PER_TILE)])

    return body


_sc_agg_deg = _sc_aggregate(with_deg=True)
_sc_agg = _sc_aggregate(with_deg=False)


BR = 2000


def _tc1_body(x_ref, wl_ref, wr_ref, y1_ref, xr_ref):
    xb = x_ref[...]
    y1_ref[...] = jnp.dot(xb, wl_ref[...], preferred_element_type=jnp.float32)
    xr_ref[...] = jnp.dot(xb, wr_ref[...], preferred_element_type=jnp.float32)


def _tc1(x, wl, wr):
    return pl.pallas_call(
        _tc1_body,
        grid=(N_NODES // BR,),
        in_specs=[
            pl.BlockSpec((BR, D_IN), lambda i: (i, 0)),
            pl.BlockSpec((D_IN, D_H), lambda i: (0, 0)),
            pl.BlockSpec((D_IN, D_H), lambda i: (0, 0)),
        ],
        out_specs=[
            pl.BlockSpec((BR, D_H), lambda i: (i, 0)),
            pl.BlockSpec((BR, D_H), lambda i: (i, 0)),
        ],
        out_shape=[
            jax.ShapeDtypeStruct((N_NODES, D_H), jnp.float32),
            jax.ShapeDtypeStruct((N_NODES, D_H), jnp.float32),
        ],
    )(x, wl, wr)


def _combine(p_ref, d_ref, res_ref, b_ref):
    agg = p_ref[0] + p_ref[1]
    deg = jnp.maximum(d_ref[..., 0] + d_ref[..., 1], 1.0)
    out = agg / deg[:, None] + b_ref[0][None, :] + res_ref[...]
    nrm = jnp.sqrt(jnp.sum(out * out, axis=1, keepdims=True))
    return out / jnp.maximum(nrm, 1e-12)


def _tc2_body(p_ref, d_ref, xr_ref, b1_ref, wl_ref, wr_ref, y2_ref, hr_ref):
    h = jnp.maximum(_combine(p_ref, d_ref, xr_ref, b1_ref), 0.0)
    y2_ref[...] = jnp.dot(h, wl_ref[...], preferred_element_type=jnp.float32)
    hr_ref[...] = jnp.dot(h, wr_ref[...], preferred_element_type=jnp.float32)


def _tc2(p, d, xr, b1, wl, wr):
    return pl.pallas_call(
        _tc2_body,
        grid=(N_NODES // BR,),
        in_specs=[
            pl.BlockSpec((NC, BR, D_H), lambda i: (0, i, 0)),
            pl.BlockSpec((BR, NC), lambda i: (i, 0)),
            pl.BlockSpec((BR, D_H), lambda i: (i, 0)),
            pl.BlockSpec((1, D_H), lambda i: (0, 0)),
            pl.BlockSpec((D_H, D_H), lambda i: (0, 0)),
            pl.BlockSpec((D_H, D_H), lambda i: (0, 0)),
        ],
        out_specs=[
            pl.BlockSpec((BR, D_H), lambda i: (i, 0)),
            pl.BlockSpec((BR, D_H), lambda i: (i, 0)),
        ],
        out_shape=[
            jax.ShapeDtypeStruct((N_NODES, D_H), jnp.float32),
            jax.ShapeDtypeStruct((N_NODES, D_H), jnp.float32),
        ],
    )(p, d, xr, b1, wl, wr)


def _tc3_body(p_ref, d_ref, hr_ref, b2_ref, wc1_ref, bc1_ref, wc2_ref,
              bc2_ref, out_ref):
    h2 = _combine(p_ref, d_ref, hr_ref, b2_ref)
    c = jnp.maximum(
        jnp.dot(h2, wc1_ref[...], preferred_element_type=jnp.float32)
        + bc1_ref[0][None, :], 0.0)
    logits = jnp.dot(c, wc2_ref[...], preferred_element_type=jnp.float32)
    out_ref[...] = logits + bc2_ref[0, 0]


def _tc3(p, d, hr, b2, wc1, bc1, wc2, bc2):
    return pl.pallas_call(
        _tc3_body,
        grid=(N_NODES // BR,),
        in_specs=[
            pl.BlockSpec((NC, BR, D_H), lambda i: (0, i, 0)),
            pl.BlockSpec((BR, NC), lambda i: (i, 0)),
            pl.BlockSpec((BR, D_H), lambda i: (i, 0)),
            pl.BlockSpec((1, D_H), lambda i: (0, 0)),
            pl.BlockSpec((D_H, D_H // 2), lambda i: (0, 0)),
            pl.BlockSpec((1, D_H // 2), lambda i: (0, 0)),
            pl.BlockSpec((D_H // 2, 1), lambda i: (0, 0)),
            pl.BlockSpec((1, 1), lambda i: (0, 0)),
        ],
        out_specs=pl.BlockSpec((BR, 1), lambda i: (i, 0)),
        out_shape=jax.ShapeDtypeStruct((N_NODES, 1), jnp.float32),
    )(p, d, hr, b2, wc1, bc1, wc2, bc2)


def kernel(x, edge_index, W1l, b1, W1r, W2l, b2, W2r, Wc1, bc1, Wc2, bc2):
    src = edge_index[0]
    dst = edge_index[1]
    pad = E_PAD - N_EDGES
    src3 = jnp.concatenate([src, jnp.zeros((pad,), jnp.int32)]).reshape(
        NW, NCHUNK, CH)
    dummy = DUMMY_DST + jnp.arange(pad, dtype=jnp.int32) % (N_ACC - N_NODES)
    dst3 = jnp.concatenate([dst, dummy]).reshape(NW, NCHUNK, CH)

    y1, xr = _tc1(x, W1l, W1r)
    agg1, deg = _sc_agg_deg(src3, dst3, y1)
    deg_t = jnp.transpose(deg[:, 0, :N_NODES])
    y2, hr = _tc2(agg1, deg_t, xr, b1.reshape(1, D_H), W2l, W2r)
    (agg2,) = _sc_agg(src3, dst3, y2)
    logits = _tc3(agg2, deg_t, hr, b2.reshape(1, D_H), Wc1,
                  bc1.reshape(1, D_H // 2), Wc2, bc2.reshape(1, 1))
    return logits.reshape(N_NODES)


# speedup vs baseline: 1.0802x; 1.0802x over previous
import functools

import jax
import jax.numpy as jnp
from jax import lax
from jax.experimental import pallas as pl
from jax.experimental.pallas import tpu as pltpu
from jax.experimental.pallas import tpu_sc as plsc

N_NODES = 10000
N_EDGES = 320000
D_IN = 128
D_H = 64

NC = 2
NS = 16
NW = NC * NS

CH = 128
NCHUNK = 80
EDGES_PER_W = CH * NCHUNK
E_PAD = EDGES_PER_W * NW
N_PAD = 10240
ROWS_PER_TILE = N_PAD // NS
N_ACC = 10112
ROWS_ACC = N_ACC // NS
DUMMY_DST = N_NODES

ZR = 79


def _sc_aggregate(with_deg):
    out_type = [jax.ShapeDtypeStruct((NC, N_NODES, D_H), jnp.float32)]
    scratch = (
        [pltpu.VMEM((NCHUNK, CH), jnp.int32),
         pltpu.VMEM((NCHUNK, CH), jnp.int32)]
        + [pltpu.VMEM((CH, D_H), jnp.float32)] * 2
        + [pltpu.VMEM((ZR, D_H), jnp.float32),
           pltpu.VMEM_SHARED((N_ACC, D_H), jnp.float32),
           pltpu.VMEM_SHARED((N_NODES, D_H), jnp.float32)]
        + [pltpu.SemaphoreType.DMA] * 2
    )
    if with_deg:
        out_type.append(jax.ShapeDtypeStruct((NC, 1, N_PAD), jnp.float32))
        scratch += [
            pltpu.VMEM((CH,), jnp.float32),
            pltpu.VMEM((ROWS_PER_TILE,), jnp.float32),
            pltpu.VMEM_SHARED((N_PAD,), jnp.float32),
            pltpu.SemaphoreType.DMA,
        ]

    mesh = plsc.VectorSubcoreMesh(core_axis_name="c", subcore_axis_name="s")

    @functools.partial(pl.kernel, out_type=out_type, mesh=mesh,
                       scratch_types=scratch,
                       compiler_params=pltpu.CompilerParams(
                           use_tc_tiling_on_sc=False))
    def body(src3, dst3, y_hbm, *rest):
        if with_deg:
            (agg_out, deg_out, src_v, dst_v, rows_a, rows_b, zrow_v, acc_sh,
             y_sh, sem_a, sem_b, ones_v, zd_v, deg_sh, dsem) = rest
        else:
            (agg_out, src_v, dst_v, rows_a, rows_b, zrow_v, acc_sh,
             y_sh, sem_a, sem_b) = rest

        cid = lax.axis_index("c")
        sid = lax.axis_index("s")
        wid = sid * NC + cid
        base_row = sid * ROWS_PER_TILE
        base_acc = sid * ROWS_ACC

        z16 = jnp.zeros((16,), jnp.float32)

        def zrow_body(r, _):
            for c4 in range(D_H // 16):
                zrow_v[r, pl.ds(c4 * 16, 16)] = z16
            return 0
        lax.fori_loop(0, ZR, zrow_body, 0)

        def zacc_body(k, _):
            pltpu.sync_copy(zrow_v, acc_sh.at[pl.ds(base_acc + k * ZR, ZR)])
            return 0
        lax.fori_loop(0, ROWS_ACC // ZR, zacc_body, 0)

        if with_deg:
            for i in range(CH // 16):
                ones_v[pl.ds(i * 16, 16)] = z16 + 1.0
            def zd_body(i, _):
                zd_v[pl.ds(i * 16, 16)] = z16
                return 0
            lax.fori_loop(0, ROWS_PER_TILE // 16, zd_body, 0)
            pltpu.sync_copy(zd_v, deg_sh.at[pl.ds(base_row, ROWS_PER_TILE)])

        pltpu.sync_copy(src3.at[wid], src_v)
        pltpu.sync_copy(dst3.at[wid], dst_v)

        nvalid_last = N_NODES - (NS - 1) * ROWS_PER_TILE

        @pl.when(sid < NS - 1)
        def _():
            pltpu.sync_copy(y_hbm.at[pl.ds(base_row, ROWS_PER_TILE)],
                            y_sh.at[pl.ds(base_row, ROWS_PER_TILE)])

        @pl.when(sid == NS - 1)
        def _():
            pltpu.sync_copy(y_hbm.at[pl.ds(base_row, nvalid_last)],
                            y_sh.at[pl.ds(base_row, nvalid_last)])

        plsc.subcore_barrier()

        pltpu.async_copy(y_sh.at[src_v.at[0]], rows_a, sem_a)

        def chunk_body(k, _):
            j0 = 2 * k
            cpb = pltpu.async_copy(y_sh.at[src_v.at[j0 + 1]], rows_b, sem_b)
            pltpu.make_async_copy(y_sh.at[src_v.at[j0]], rows_a, sem_a).wait()
            if with_deg:
                d0 = pltpu.async_copy(ones_v, deg_sh.at[dst_v.at[j0]],
                                      dsem, add=True)
            pltpu.sync_copy(rows_a, acc_sh.at[dst_v.at[j0]], add=True)

            @pl.when(j0 + 2 < NCHUNK)
            def _():
                pltpu.async_copy(y_sh.at[src_v.at[j0 + 2]], rows_a, sem_a)

            cpb.wait()
            if with_deg:
                d1 = pltpu.async_copy(ones_v, deg_sh.at[dst_v.at[j0 + 1]],
                                      dsem, add=True)
            pltpu.sync_copy(rows_b, acc_sh.at[dst_v.at[j0 + 1]], add=True)
            if with_deg:
                d0.wait()
                d1.wait()
            return 0
        lax.fori_loop(0, NCHUNK // 2, chunk_body, 0)

        plsc.subcore_barrier()

        nacc_last = N_NODES - (NS - 1) * ROWS_ACC

        @pl.when(sid < NS - 1)
        def _():
            pltpu.sync_copy(acc_sh.at[pl.ds(base_acc, ROWS_ACC)],
                            agg_out.at[cid, pl.ds(base_acc, ROWS_ACC)])

        @pl.when(sid == NS - 1)
        def _():
            pltpu.sync_copy(acc_sh.at[pl.ds(base_acc, nacc_last)],
                            agg_out.at[cid, pl.ds(base_acc, nacc_last)])

        if with_deg:
            pltpu.sync_copy(deg_sh.at[pl.ds(base_row, ROWS_PER_TILE)],
                            deg_out.at[cid, 0, pl.ds(base_row, ROWS_PER_TILE)])

    return body


_sc_agg_deg = _sc_aggregate(with_deg=True)
_sc_agg = _sc_aggregate(with_deg=False)


BR = 2000


def _tc1_body(x_ref, wl_ref, wr_ref, y1_ref, xr_ref):
    xb = x_ref[...]
    y1_ref[...] = jnp.dot(xb, wl_ref[...], preferred_element_type=jnp.float32)
    xr_ref[...] = jnp.dot(xb, wr_ref[...], preferred_element_type=jnp.float32)


def _tc1(x, wl, wr):
    return pl.pallas_call(
        _tc1_body,
        grid=(N_NODES // BR,),
        in_specs=[
            pl.BlockSpec((BR, D_IN), lambda i: (i, 0)),
            pl.BlockSpec((D_IN, D_H), lambda i: (0, 0)),
            pl.BlockSpec((D_IN, D_H), lambda i: (0, 0)),
        ],
        out_specs=[
            pl.BlockSpec((BR, D_H), lambda i: (i, 0)),
            pl.BlockSpec((BR, D_H), lambda i: (i, 0)),
        ],
        out_shape=[
            jax.ShapeDtypeStruct((N_NODES, D_H), jnp.float32),
            jax.ShapeDtypeStruct((N_NODES, D_H), jnp.float32),
        ],
    )(x, wl, wr)


def _combine(p_ref, d_ref, res_ref, b_ref):
    agg = p_ref[0] + p_ref[1]
    deg = jnp.maximum(d_ref[..., 0] + d_ref[..., 1], 1.0)
    out = agg / deg[:, None] + b_ref[0][None, :] + res_ref[...]
    nrm = jnp.sqrt(jnp.sum(out * out, axis=1, keepdims=True))
    return out / jnp.maximum(nrm, 1e-12)


def _tc2_body(p_ref, d_ref, xr_ref, b1_ref, wl_ref, wr_ref, y2_ref, hr_ref):
    h = jnp.maximum(_combine(p_ref, d_ref, xr_ref, b1_ref), 0.0)
    y2_ref[...] = jnp.dot(h, wl_ref[...], preferred_element_type=jnp.float32)
    hr_ref[...] = jnp.dot(h, wr_ref[...], preferred_element_type=jnp.float32)


def _tc2(p, d, xr, b1, wl, wr):
    return pl.pallas_call(
        _tc2_body,
        grid=(N_NODES // BR,),
        in_specs=[
            pl.BlockSpec((NC, BR, D_H), lambda i: (0, i, 0)),
            pl.BlockSpec((BR, NC), lambda i: (i, 0)),
            pl.BlockSpec((BR, D_H), lambda i: (i, 0)),
            pl.BlockSpec((1, D_H), lambda i: (0, 0)),
            pl.BlockSpec((D_H, D_H), lambda i: (0, 0)),
            pl.BlockSpec((D_H, D_H), lambda i: (0, 0)),
        ],
        out_specs=[
            pl.BlockSpec((BR, D_H), lambda i: (i, 0)),
            pl.BlockSpec((BR, D_H), lambda i: (i, 0)),
        ],
        out_shape=[
            jax.ShapeDtypeStruct((N_NODES, D_H), jnp.float32),
            jax.ShapeDtypeStruct((N_NODES, D_H), jnp.float32),
        ],
    )(p, d, xr, b1, wl, wr)


def _tc3_body(p_ref, d_ref, hr_ref, b2_ref, wc1_ref, bc1_ref, wc2_ref,
              bc2_ref, out_ref):
    h2 = _combine(p_ref, d_ref, hr_ref, b2_ref)
    c = jnp.maximum(
        jnp.dot(h2, wc1_ref[...], preferred_element_type=jnp.float32)
        + bc1_ref[0][None, :], 0.0)
    logits = jnp.dot(c, wc2_ref[...], preferred_element_type=jnp.float32)
    out_ref[...] = logits + bc2_ref[0, 0]


def _tc3(p, d, hr, b2, wc1, bc1, wc2, bc2):
    return pl.pallas_call(
        _tc3_body,
        grid=(N_NODES // BR,),
        in_specs=[
            pl.BlockSpec((NC, BR, D_H), lambda i: (0, i, 0)),
            pl.BlockSpec((BR, NC), lambda i: (i, 0)),
            pl.BlockSpec((BR, D_H), lambda i: (i, 0)),
            pl.BlockSpec((1, D_H), lambda i: (0, 0)),
            pl.BlockSpec((D_H, D_H // 2), lambda i: (0, 0)),
            pl.BlockSpec((1, D_H // 2), lambda i: (0, 0)),
            pl.BlockSpec((D_H // 2, 1), lambda i: (0, 0)),
            pl.BlockSpec((1, 1), lambda i: (0, 0)),
        ],
        out_specs=pl.BlockSpec((BR, 1), lambda i: (i, 0)),
        out_shape=jax.ShapeDtypeStruct((N_NODES, 1), jnp.float32),
    )(p, d, hr, b2, wc1, bc1, wc2, bc2)


def kernel(x, edge_index, W1l, b1, W1r, W2l, b2, W2r, Wc1, bc1, Wc2, bc2):
    src = edge_index[0]
    dst = edge_index[1]
    pad = E_PAD - N_EDGES
    src3 = jnp.concatenate([src, jnp.zeros((pad,), jnp.int32)]).reshape(
        NW, NCHUNK, CH)
    dummy = DUMMY_DST + jnp.arange(pad, dtype=jnp.int32) % (N_ACC - N_NODES)
    dst3 = jnp.concatenate([dst, dummy]).reshape(NW, NCHUNK, CH)

    y1, xr = _tc1(x, W1l, W1r)
    agg1, deg = _sc_agg_deg(src3, dst3, y1)
    deg_t = jnp.transpose(deg[:, 0, :N_NODES])
    y2, hr = _tc2(agg1, deg_t, xr, b1.reshape(1, D_H), W2l, W2r)
    (agg2,) = _sc_agg(src3, dst3, y2)
    logits = _tc3(agg2, deg_t, hr, b2.reshape(1, D_H), Wc1,
                  bc1.reshape(1, D_H // 2), Wc2, bc2.reshape(1, 1))
    return logits.reshape(N_NODES)


# speedup vs baseline: 1.0872x; 1.0065x over previous
import functools

import jax
import jax.numpy as jnp
from jax import lax
from jax.experimental import pallas as pl
from jax.experimental.pallas import tpu as pltpu
from jax.experimental.pallas import tpu_sc as plsc

N_NODES = 10000
N_EDGES = 320000
D_IN = 128
D_H = 64

NC = 2
NS = 16
NW = NC * NS

CH = 200
NCHUNK = 50
EDGES_PER_W = CH * NCHUNK
N_PAD = 10240
ROWS_PER_TILE = N_PAD // NS
N_ACC = 10112
ROWS_ACC = N_ACC // NS

ZR = 79
CH_PAD = 208


def _sc_aggregate(with_deg):
    out_type = [jax.ShapeDtypeStruct((NC, N_NODES, D_H), jnp.float32)]
    scratch = (
        [pltpu.VMEM((NCHUNK, CH), jnp.int32),
         pltpu.VMEM((NCHUNK, CH), jnp.int32)]
        + [pltpu.VMEM((CH, D_H), jnp.float32)] * 2
        + [pltpu.VMEM_SHARED((N_ACC, D_H), jnp.float32),
           pltpu.VMEM_SHARED((N_NODES, D_H), jnp.float32)]
        + [pltpu.SemaphoreType.DMA] * 2
    )
    if with_deg:
        out_type.append(jax.ShapeDtypeStruct((NC, 1, N_PAD), jnp.float32))
        scratch += [
            pltpu.VMEM((CH_PAD,), jnp.float32),
            pltpu.VMEM((ROWS_PER_TILE,), jnp.float32),
            pltpu.VMEM_SHARED((N_PAD,), jnp.float32),
            pltpu.SemaphoreType.DMA,
        ]

    mesh = plsc.VectorSubcoreMesh(core_axis_name="c", subcore_axis_name="s")

    @functools.partial(pl.kernel, out_type=out_type, mesh=mesh,
                       scratch_types=scratch,
                       compiler_params=pltpu.CompilerParams(
                           use_tc_tiling_on_sc=False))
    def body(src3, dst3, y_hbm, *rest):
        if with_deg:
            (agg_out, deg_out, src_v, dst_v, rows_a, rows_b, acc_sh,
             y_sh, sem_a, sem_b, ones_v, zd_v, deg_sh, dsem) = rest
        else:
            (agg_out, src_v, dst_v, rows_a, rows_b, acc_sh,
             y_sh, sem_a, sem_b) = rest

        cid = lax.axis_index("c")
        sid = lax.axis_index("s")
        wid = sid * NC + cid
        base_row = sid * ROWS_PER_TILE
        base_acc = sid * ROWS_ACC

        z16 = jnp.zeros((16,), jnp.float32)

        def zrow_body(r, _):
            for c4 in range(D_H // 16):
                rows_a[r, pl.ds(c4 * 16, 16)] = z16
            return 0
        lax.fori_loop(0, ZR, zrow_body, 0)

        def zacc_body(k, _):
            pltpu.sync_copy(rows_a.at[pl.ds(0, ZR)],
                            acc_sh.at[pl.ds(base_acc + k * ZR, ZR)])
            return 0
        lax.fori_loop(0, ROWS_ACC // ZR, zacc_body, 0)

        if with_deg:
            for i in range(CH_PAD // 16):
                ones_v[pl.ds(i * 16, 16)] = z16 + 1.0
            def zd_body(i, _):
                zd_v[pl.ds(i * 16, 16)] = z16
                return 0
            lax.fori_loop(0, ROWS_PER_TILE // 16, zd_body, 0)
            pltpu.sync_copy(zd_v, deg_sh.at[pl.ds(base_row, ROWS_PER_TILE)])

        pltpu.sync_copy(src3.at[wid], src_v)
        pltpu.sync_copy(dst3.at[wid], dst_v)

        nvalid_last = N_NODES - (NS - 1) * ROWS_PER_TILE

        @pl.when(sid < NS - 1)
        def _():
            pltpu.sync_copy(y_hbm.at[pl.ds(base_row, ROWS_PER_TILE)],
                            y_sh.at[pl.ds(base_row, ROWS_PER_TILE)])

        @pl.when(sid == NS - 1)
        def _():
            pltpu.sync_copy(y_hbm.at[pl.ds(base_row, nvalid_last)],
                            y_sh.at[pl.ds(base_row, nvalid_last)])

        plsc.subcore_barrier()

        pltpu.async_copy(y_sh.at[src_v.at[0]], rows_a, sem_a)

        def chunk_body(k, _):
            j0 = 2 * k
            cpb = pltpu.async_copy(y_sh.at[src_v.at[j0 + 1]], rows_b, sem_b)
            pltpu.make_async_copy(y_sh.at[src_v.at[j0]], rows_a, sem_a).wait()
            if with_deg:
                d0 = pltpu.async_copy(ones_v.at[pl.ds(0, CH)],
                                      deg_sh.at[dst_v.at[j0]],
                                      dsem, add=True)
            pltpu.sync_copy(rows_a, acc_sh.at[dst_v.at[j0]], add=True)

            @pl.when(j0 + 2 < NCHUNK)
            def _():
                pltpu.async_copy(y_sh.at[src_v.at[j0 + 2]], rows_a, sem_a)

            cpb.wait()
            if with_deg:
                d1 = pltpu.async_copy(ones_v.at[pl.ds(0, CH)],
                                      deg_sh.at[dst_v.at[j0 + 1]],
                                      dsem, add=True)
            pltpu.sync_copy(rows_b, acc_sh.at[dst_v.at[j0 + 1]], add=True)
            if with_deg:
                d0.wait()
                d1.wait()
            return 0
        lax.fori_loop(0, NCHUNK // 2, chunk_body, 0)

        plsc.subcore_barrier()

        nacc_last = N_NODES - (NS - 1) * ROWS_ACC

        @pl.when(sid < NS - 1)
        def _():
            pltpu.sync_copy(acc_sh.at[pl.ds(base_acc, ROWS_ACC)],
                            agg_out.at[cid, pl.ds(base_acc, ROWS_ACC)])

        @pl.when(sid == NS - 1)
        def _():
            pltpu.sync_copy(acc_sh.at[pl.ds(base_acc, nacc_last)],
                            agg_out.at[cid, pl.ds(base_acc, nacc_last)])

        if with_deg:
            pltpu.sync_copy(deg_sh.at[pl.ds(base_row, ROWS_PER_TILE)],
                            deg_out.at[cid, 0, pl.ds(base_row, ROWS_PER_TILE)])

    return body


_sc_agg_deg = _sc_aggregate(with_deg=True)
_sc_agg = _sc_aggregate(with_deg=False)


BR = 2000


def _tc1_body(x_ref, wl_ref, wr_ref, y1_ref, xr_ref):
    xb = x_ref[...]
    y1_ref[...] = jnp.dot(xb, wl_ref[...], preferred_element_type=jnp.float32)
    xr_ref[...] = jnp.dot(xb, wr_ref[...], preferred_element_type=jnp.float32)


def _tc1(x, wl, wr):
    return pl.pallas_call(
        _tc1_body,
        grid=(N_NODES // BR,),
        in_specs=[
            pl.BlockSpec((BR, D_IN), lambda i: (i, 0)),
            pl.BlockSpec((D_IN, D_H), lambda i: (0, 0)),
            pl.BlockSpec((D_IN, D_H), lambda i: (0, 0)),
        ],
        out_specs=[
            pl.BlockSpec((BR, D_H), lambda i: (i, 0)),
            pl.BlockSpec((BR, D_H), lambda i: (i, 0)),
        ],
        out_shape=[
            jax.ShapeDtypeStruct((N_NODES, D_H), jnp.float32),
            jax.ShapeDtypeStruct((N_NODES, D_H), jnp.float32),
        ],
    )(x, wl, wr)


def _combine(p_ref, d_ref, res_ref, b_ref):
    agg = p_ref[0] + p_ref[1]
    deg = jnp.maximum(d_ref[..., 0] + d_ref[..., 1], 1.0)
    out = agg / deg[:, None] + b_ref[0][None, :] + res_ref[...]
    nrm = jnp.sqrt(jnp.sum(out * out, axis=1, keepdims=True))
    return out / jnp.maximum(nrm, 1e-12)


def _tc2_body(p_ref, d_ref, xr_ref, b1_ref, wl_ref, wr_ref, y2_ref, hr_ref):
    h = jnp.maximum(_combine(p_ref, d_ref, xr_ref, b1_ref), 0.0)
    y2_ref[...] = jnp.dot(h, wl_ref[...], preferred_element_type=jnp.float32)
    hr_ref[...] = jnp.dot(h, wr_ref[...], preferred_element_type=jnp.float32)


def _tc2(p, d, xr, b1, wl, wr):
    return pl.pallas_call(
        _tc2_body,
        grid=(N_NODES // BR,),
        in_specs=[
            pl.BlockSpec((NC, BR, D_H), lambda i: (0, i, 0)),
            pl.BlockSpec((BR, NC), lambda i: (i, 0)),
            pl.BlockSpec((BR, D_H), lambda i: (i, 0)),
            pl.BlockSpec((1, D_H), lambda i: (0, 0)),
            pl.BlockSpec((D_H, D_H), lambda i: (0, 0)),
            pl.BlockSpec((D_H, D_H), lambda i: (0, 0)),
        ],
        out_specs=[
            pl.BlockSpec((BR, D_H), lambda i: (i, 0)),
            pl.BlockSpec((BR, D_H), lambda i: (i, 0)),
        ],
        out_shape=[
            jax.ShapeDtypeStruct((N_NODES, D_H), jnp.float32),
            jax.ShapeDtypeStruct((N_NODES, D_H), jnp.float32),
        ],
    )(p, d, xr, b1, wl, wr)


def _tc3_body(p_ref, d_ref, hr_ref, b2_ref, wc1_ref, bc1_ref, wc2_ref,
              bc2_ref, out_ref):
    h2 = _combine(p_ref, d_ref, hr_ref, b2_ref)
    c = jnp.maximum(
        jnp.dot(h2, wc1_ref[...], preferred_element_type=jnp.float32)
        + bc1_ref[0][None, :], 0.0)
    logits = jnp.dot(c, wc2_ref[...], preferred_element_type=jnp.float32)
    out_ref[...] = logits + bc2_ref[0, 0]


def _tc3(p, d, hr, b2, wc1, bc1, wc2, bc2):
    return pl.pallas_call(
        _tc3_body,
        grid=(N_NODES // BR,),
        in_specs=[
            pl.BlockSpec((NC, BR, D_H), lambda i: (0, i, 0)),
            pl.BlockSpec((BR, NC), lambda i: (i, 0)),
            pl.BlockSpec((BR, D_H), lambda i: (i, 0)),
            pl.BlockSpec((1, D_H), lambda i: (0, 0)),
            pl.BlockSpec((D_H, D_H // 2), lambda i: (0, 0)),
            pl.BlockSpec((1, D_H // 2), lambda i: (0, 0)),
            pl.BlockSpec((D_H // 2, 1), lambda i: (0, 0)),
            pl.BlockSpec((1, 1), lambda i: (0, 0)),
        ],
        out_specs=pl.BlockSpec((BR, 1), lambda i: (i, 0)),
        out_shape=jax.ShapeDtypeStruct((N_NODES, 1), jnp.float32),
    )(p, d, hr, b2, wc1, bc1, wc2, bc2)


def kernel(x, edge_index, W1l, b1, W1r, W2l, b2, W2r, Wc1, bc1, Wc2, bc2):
    src3 = edge_index[0].reshape(NW, NCHUNK, CH)
    dst3 = edge_index[1].reshape(NW, NCHUNK, CH)

    y1, xr = _tc1(x, W1l, W1r)
    agg1, deg = _sc_agg_deg(src3, dst3, y1)
    deg_t = jnp.transpose(deg[:, 0, :N_NODES])
    y2, hr = _tc2(agg1, deg_t, xr, b1.reshape(1, D_H), W2l, W2r)
    (agg2,) = _sc_agg(src3, dst3, y2)
    logits = _tc3(agg2, deg_t, hr, b2.reshape(1, D_H), Wc1,
                  bc1.reshape(1, D_H // 2), Wc2, bc2.reshape(1, 1))
    return logits.reshape(N_NODES)
